# baseline (speedup 1.0000x reference)
import functools

import jax
import jax.numpy as jnp
from jax import lax
from jax.experimental import pallas as pl
from jax.experimental.pallas import tpu as pltpu
from jax.experimental.pallas import tpu_sc as plsc

_NUM_CORES = 2
_NUM_SUBCORES = 16
_NUM_WORKERS = _NUM_CORES * _NUM_SUBCORES
_CHUNK = 256
_NBUF = 3
_LANES = 128
_VB = 128


def _mesh():
    return plsc.VectorSubcoreMesh(core_axis_name="c", subcore_axis_name="s")


@functools.partial(jax.jit, static_argnames=("vocab", "embed"))
def _transpose_sc(table_t, *, vocab, embed):
    n_full = vocab // _VB
    max_blocks = -(-n_full // _NUM_WORKERS)
    n_pairs = -(-max_blocks // 2)

    @functools.partial(
        pl.kernel,
        out_type=jax.ShapeDtypeStruct((vocab, _LANES), jnp.float32),
        mesh=_mesh(),
        compiler_params=pltpu.CompilerParams(
            use_tc_tiling_on_sc=True, needs_layout_passes=False
        ),
        scratch_types=[
            pltpu.VMEM((2, embed, _VB), jnp.float32),
            pltpu.VMEM((2, _VB, _LANES), jnp.float32),
            pltpu.SemaphoreType.DMA((2,)),
            pltpu.SemaphoreType.DMA((2,)),
        ],
    )
    def k(tt_hbm, out_hbm, inb, outb, isem, osem):
        wid = lax.axis_index("s") * _NUM_CORES + lax.axis_index("c")
        lane_iota = lax.iota(jnp.int32, 16)

        def transpose_block(buf, blk):
            def col(c, carry):
                for g in range(embed // 16):
                    v = plsc.load_gather(
                        inb.at[buf], [g * 16 + lane_iota, jnp.full((16,), c, jnp.int32)]
                    )
                    outb[buf, c, pl.ds(g * 16, 16)] = v
                return carry

            lax.fori_loop(0, _VB, col, 0)

        def start_in(buf, blk):
            return pltpu.async_copy(
                tt_hbm.at[:, pl.ds(blk * _VB, _VB)], inb.at[buf], isem.at[buf]
            )

        def start_out(buf, blk):
            return pltpu.async_copy(
                outb.at[buf], out_hbm.at[pl.ds(blk * _VB, _VB)], osem.at[buf]
            )

        def do_block(buf, blk):
            pltpu.make_async_copy(
                tt_hbm.at[:, pl.ds(0, _VB)], inb.at[buf], isem.at[buf]
            ).wait()
            transpose_block(buf, blk)
            start_out(buf, blk)

        def wait_out(buf):
            pltpu.make_async_copy(
                outb.at[buf], out_hbm.at[pl.ds(0, _VB)], osem.at[buf]
            ).wait()

        def blk_of(i):
            return wid + i * _NUM_WORKERS

        n_mine = n_full // _NUM_WORKERS + jnp.where(
            wid < n_full % _NUM_WORKERS, 1, 0
        )

        @pl.when(n_mine > 0)
        def _():
            start_in(0, blk_of(0))

        @pl.when(n_mine > 1)
        def _():
            start_in(1, blk_of(1))

        def pair(j, carry):
            i0 = 2 * j

            @pl.when(i0 < n_mine)
            def _():
                do_block(0, blk_of(i0))

                @pl.when(i0 + 2 < n_mine)
                def _():
                    wait_out(0)
                    start_in(0, blk_of(i0 + 2))

            @pl.when(i0 + 1 < n_mine)
            def _():
                do_block(1, blk_of(i0 + 1))

                @pl.when(i0 + 3 < n_mine)
                def _():
                    wait_out(1)
                    start_in(1, blk_of(i0 + 3))

            return carry

        lax.fori_loop(0, n_pairs, pair, 0)

        @pl.when(n_mine > 0)
        def _():
            wait_out(0)

        @pl.when(n_mine > 1)
        def _():
            wait_out(1)

    return k(table_t)


@functools.partial(jax.jit, static_argnames=("n_chunks", "embed"))
def _gather_sc(idx, table_pad, *, n_chunks, embed):
    @functools.partial(
        pl.kernel,
        out_type=jax.ShapeDtypeStruct(
            (_NUM_WORKERS, n_chunks, _CHUNK, embed), jnp.float32
        ),
        mesh=_mesh(),
        compiler_params=pltpu.CompilerParams(use_tc_tiling_on_sc=False),
        scratch_types=[
            pltpu.VMEM((n_chunks, _CHUNK), jnp.int32),
            pltpu.VMEM((_NBUF, _CHUNK, _LANES), jnp.float32),
            pltpu.SemaphoreType.DMA((_NBUF,)),
            pltpu.SemaphoreType.DMA((_NBUF,)),
        ],
    )
    def k(idx_hbm, table_hbm, out_hbm, idx_v, rows_v, gsem, osem):
        wid = lax.axis_index("s") * _NUM_CORES + lax.axis_index("c")
        pltpu.sync_copy(idx_hbm.at[wid], idx_v)

        gathers = [None] * n_chunks
        outs = [None] * n_chunks

        def start_out(g):
            b = g % _NBUF
            return pltpu.async_copy(
                rows_v.at[b, slice(None), pl.ds(0, embed)],
                out_hbm.at[wid, g],
                osem.at[b],
            )

        for g in range(n_chunks):
            b = g % _NBUF
            if g >= _NBUF:
                outs[g - _NBUF].wait()
            gathers[g] = pltpu.async_copy(
                table_hbm.at[idx_v.at[g]], rows_v.at[b], gsem.at[b]
            )
            if g >= 1:
                gathers[g - 1].wait()
                outs[g - 1] = start_out(g - 1)
        gathers[n_chunks - 1].wait()
        outs[n_chunks - 1] = start_out(n_chunks - 1)
        for g in range(max(0, n_chunks - _NBUF), n_chunks):
            outs[g].wait()

    return k(idx, table_pad)


def kernel(token_ids, table):
    b, l = token_ids.shape
    vocab, embed = table.shape
    n = b * l
    assert n % (_NUM_WORKERS * _CHUNK) == 0
    n_chunks = n // (_NUM_WORKERS * _CHUNK)
    idx = token_ids.astype(jnp.int32).reshape(_NUM_WORKERS, n_chunks, _CHUNK)
    table_pad = _transpose_sc(table.T, vocab=vocab, embed=embed)
    aligned = (vocab // _VB) * _VB
    if aligned < vocab:
        tail = jnp.pad(table[aligned:], ((0, 0), (0, _LANES - embed)))
        table_pad = lax.dynamic_update_slice(table_pad, tail, (aligned, 0))
    out = _gather_sc(idx, table_pad, n_chunks=n_chunks, embed=embed)
    return out.reshape(b, l, embed)


# speedup vs baseline: 2.2641x; 2.2641x over previous
import functools

import jax
import jax.numpy as jnp
from jax import lax
from jax.experimental import pallas as pl
from jax.experimental.pallas import tpu as pltpu
from jax.experimental.pallas import tpu_sc as plsc

_NUM_CORES = 2
_NUM_SUBCORES = 16
_NUM_WORKERS = _NUM_CORES * _NUM_SUBCORES
_CHUNK = 256
_NBUF = 3
_LANES = 128
_BLK = 2048


def _pad_body(t_ref, eye_ref, o_ref):
    o_ref[...] = lax.dot_general(
        t_ref[...],
        eye_ref[...],
        (((0,), (0,)), ((), ())),
        preferred_element_type=jnp.float32,
        precision=lax.Precision.HIGHEST,
    )


@functools.partial(jax.jit, static_argnames=("vocab", "embed"))
def _pad_tc(table_t, eyepad, *, vocab, embed):
    return pl.pallas_call(
        _pad_body,
        grid=(pl.cdiv(vocab, _BLK),),
        in_specs=[
            pl.BlockSpec((embed, _BLK), lambda i: (0, i)),
            pl.BlockSpec((embed, _LANES), lambda i: (0, 0)),
        ],
        out_specs=pl.BlockSpec((_BLK, _LANES), lambda i: (i, 0)),
        out_shape=jax.ShapeDtypeStruct((vocab, _LANES), jnp.float32),
    )(table_t, eyepad)


@functools.partial(jax.jit, static_argnames=("n_chunks", "embed"))
def _gather_sc(idx, table_pad, *, n_chunks, embed):
    mesh = plsc.VectorSubcoreMesh(core_axis_name="c", subcore_axis_name="s")

    @functools.partial(
        pl.kernel,
        out_type=jax.ShapeDtypeStruct(
            (_NUM_WORKERS, n_chunks, _CHUNK, embed), jnp.float32
        ),
        mesh=mesh,
        compiler_params=pltpu.CompilerParams(use_tc_tiling_on_sc=False),
        scratch_types=[
            pltpu.VMEM((n_chunks, _CHUNK), jnp.int32),
            pltpu.VMEM((_NBUF, _CHUNK, _LANES), jnp.float32),
            pltpu.SemaphoreType.DMA((_NBUF,)),
            pltpu.SemaphoreType.DMA((_NBUF,)),
        ],
    )
    def k(idx_hbm, table_hbm, out_hbm, idx_v, rows_v, gsem, osem):
        wid = lax.axis_index("s") * _NUM_CORES + lax.axis_index("c")
        pltpu.sync_copy(idx_hbm.at[wid], idx_v)

        gathers = [None] * n_chunks
        outs = [None] * n_chunks

        def start_out(g):
            b = g % _NBUF
            return pltpu.async_copy(
                rows_v.at[b, slice(None), pl.ds(0, embed)],
                out_hbm.at[wid, g],
                osem.at[b],
            )

        for g in range(n_chunks):
            b = g % _NBUF
            if g >= _NBUF:
                outs[g - _NBUF].wait()
            gathers[g] = pltpu.async_copy(
                table_hbm.at[idx_v.at[g]], rows_v.at[b], gsem.at[b]
            )
            if g >= 1:
                gathers[g - 1].wait()
                outs[g - 1] = start_out(g - 1)
        gathers[n_chunks - 1].wait()
        outs[n_chunks - 1] = start_out(n_chunks - 1)
        for g in range(max(0, n_chunks - _NBUF), n_chunks):
            outs[g].wait()

    return k(idx, table_pad)


def kernel(token_ids, table):
    b, l = token_ids.shape
    vocab, embed = table.shape
    n = b * l
    assert n % (_NUM_WORKERS * _CHUNK) == 0
    n_chunks = n // (_NUM_WORKERS * _CHUNK)
    idx = token_ids.astype(jnp.int32).reshape(_NUM_WORKERS, n_chunks, _CHUNK)
    eyepad = jnp.eye(embed, _LANES, dtype=jnp.float32)
    table_pad = _pad_tc(table.T, eyepad, vocab=vocab, embed=embed)
    out = _gather_sc(idx, table_pad, n_chunks=n_chunks, embed=embed)
    return out.reshape(b, l, embed)


# speedup vs baseline: 2.7687x; 1.2229x over previous
import functools

import jax
import jax.numpy as jnp
from jax import lax
from jax.experimental import pallas as pl
from jax.experimental.pallas import tpu as pltpu
from jax.experimental.pallas import tpu_sc as plsc

_NUM_CORES = 2
_NUM_SUBCORES = 16
_NUM_WORKERS = _NUM_CORES * _NUM_SUBCORES
_CHUNK = 256
_NBUF = 3
_LANES = 128
_BLK = 2048


def _pad_body(t_ref, eye_ref, o_ref):
    del eye_ref
    xt = t_ref[...].T
    o_ref[...] = jnp.concatenate(
        [xt, jnp.zeros((xt.shape[0], _LANES - xt.shape[1]), jnp.float32)], axis=1
    )


@functools.partial(jax.jit, static_argnames=("vocab", "embed"))
def _pad_tc(table_t, eyepad, *, vocab, embed):
    return pl.pallas_call(
        _pad_body,
        grid=(pl.cdiv(vocab, _BLK),),
        in_specs=[
            pl.BlockSpec((embed, _BLK), lambda i: (0, i)),
            pl.BlockSpec((embed, _LANES), lambda i: (0, 0)),
        ],
        out_specs=pl.BlockSpec((_BLK, _LANES), lambda i: (i, 0)),
        out_shape=jax.ShapeDtypeStruct((vocab, _LANES), jnp.float32),
    )(table_t, eyepad)


@functools.partial(jax.jit, static_argnames=("n_chunks", "embed"))
def _gather_sc(idx, table_pad, *, n_chunks, embed):
    mesh = plsc.VectorSubcoreMesh(core_axis_name="c", subcore_axis_name="s")

    @functools.partial(
        pl.kernel,
        out_type=jax.ShapeDtypeStruct(
            (_NUM_WORKERS, n_chunks, _CHUNK, embed), jnp.float32
        ),
        mesh=mesh,
        compiler_params=pltpu.CompilerParams(use_tc_tiling_on_sc=False),
        scratch_types=[
            pltpu.VMEM((n_chunks, _CHUNK), jnp.int32),
            pltpu.VMEM((_NBUF, _CHUNK, _LANES), jnp.float32),
            pltpu.SemaphoreType.DMA((_NBUF,)),
            pltpu.SemaphoreType.DMA((_NBUF,)),
        ],
    )
    def k(idx_hbm, table_hbm, out_hbm, idx_v, rows_v, gsem, osem):
        wid = lax.axis_index("s") * _NUM_CORES + lax.axis_index("c")
        pltpu.sync_copy(idx_hbm.at[wid], idx_v)

        gathers = [None] * n_chunks
        outs = [None] * n_chunks

        def start_out(g):
            b = g % _NBUF
            return pltpu.async_copy(
                rows_v.at[b, slice(None), pl.ds(0, embed)],
                out_hbm.at[wid, g],
                osem.at[b],
            )

        for g in range(n_chunks):
            b = g % _NBUF
            if g >= _NBUF:
                outs[g - _NBUF].wait()
            gathers[g] = pltpu.async_copy(
                table_hbm.at[idx_v.at[g]], rows_v.at[b], gsem.at[b]
            )
            if g >= 1:
                gathers[g - 1].wait()
                outs[g - 1] = start_out(g - 1)
        gathers[n_chunks - 1].wait()
        outs[n_chunks - 1] = start_out(n_chunks - 1)
        for g in range(max(0, n_chunks - _NBUF), n_chunks):
            outs[g].wait()

    return k(idx, table_pad)


def kernel(token_ids, table):
    b, l = token_ids.shape
    vocab, embed = table.shape
    n = b * l
    assert n % (_NUM_WORKERS * _CHUNK) == 0
    n_chunks = n // (_NUM_WORKERS * _CHUNK)
    idx = token_ids.astype(jnp.int32).reshape(_NUM_WORKERS, n_chunks, _CHUNK)
    eyepad = jnp.eye(embed, _LANES, dtype=jnp.float32)
    table_pad = _pad_tc(table.T, eyepad, vocab=vocab, embed=embed)
    out = _gather_sc(idx, table_pad, n_chunks=n_chunks, embed=embed)
    return out.reshape(b, l, embed)


# speedup vs baseline: 3.9342x; 1.4209x over previous
import functools

import jax
import jax.numpy as jnp
from jax import lax
from jax.experimental import pallas as pl
from jax.experimental.pallas import tpu as pltpu
from jax.experimental.pallas import tpu_sc as plsc

_NUM_CORES = 2
_NUM_SUBCORES = 16
_NUM_WORKERS = _NUM_CORES * _NUM_SUBCORES
_CHUNK = 256
_NBUF = 3
_LANES = 128
_BLK = 8192


def _pad_body(t_ref, eye_ref, o_ref):
    del eye_ref
    xt = t_ref[...].T
    o_ref[...] = jnp.concatenate(
        [xt, jnp.zeros((xt.shape[0], _LANES - xt.shape[1]), jnp.float32)], axis=1
    )


@functools.partial(jax.jit, static_argnames=("vocab", "embed"))
def _pad_tc(table_t, eyepad, *, vocab, embed):
    return pl.pallas_call(
        _pad_body,
        grid=(pl.cdiv(vocab, _BLK),),
        in_specs=[
            pl.BlockSpec((embed, _BLK), lambda i: (0, i)),
            pl.BlockSpec((embed, _LANES), lambda i: (0, 0)),
        ],
        out_specs=pl.BlockSpec((_BLK, _LANES), lambda i: (i, 0)),
        out_shape=jax.ShapeDtypeStruct((vocab, _LANES), jnp.float32),
    )(table_t, eyepad)


@functools.partial(jax.jit, static_argnames=("n_chunks", "embed"))
def _gather_sc(idx, table_pad, *, n_chunks, embed):
    mesh = plsc.VectorSubcoreMesh(core_axis_name="c", subcore_axis_name="s")

    @functools.partial(
        pl.kernel,
        out_type=jax.ShapeDtypeStruct(
            (_NUM_WORKERS, n_chunks, _CHUNK, embed), jnp.float32
        ),
        mesh=mesh,
        compiler_params=pltpu.CompilerParams(use_tc_tiling_on_sc=False),
        scratch_types=[
            pltpu.VMEM((n_chunks, _CHUNK), jnp.int32),
            pltpu.VMEM((_NBUF, _CHUNK, _LANES), jnp.float32),
            pltpu.SemaphoreType.DMA((_NBUF,)),
            pltpu.SemaphoreType.DMA((_NBUF,)),
        ],
    )
    def k(idx_hbm, table_hbm, out_hbm, idx_v, rows_v, gsem, osem):
        wid = lax.axis_index("s") * _NUM_CORES + lax.axis_index("c")
        pltpu.sync_copy(idx_hbm.at[wid], idx_v)

        gathers = [None] * n_chunks
        outs = [None] * n_chunks

        def start_out(g):
            b = g % _NBUF
            return pltpu.async_copy(
                rows_v.at[b, slice(None), pl.ds(0, embed)],
                out_hbm.at[wid, g],
                osem.at[b],
            )

        for g in range(n_chunks):
            b = g % _NBUF
            if g >= _NBUF:
                outs[g - _NBUF].wait()
            gathers[g] = pltpu.async_copy(
                table_hbm.at[idx_v.at[g]], rows_v.at[b], gsem.at[b]
            )
            if g >= 1:
                gathers[g - 1].wait()
                outs[g - 1] = start_out(g - 1)
        gathers[n_chunks - 1].wait()
        outs[n_chunks - 1] = start_out(n_chunks - 1)
        for g in range(max(0, n_chunks - _NBUF), n_chunks):
            outs[g].wait()

    return k(idx, table_pad)


def kernel(token_ids, table):
    b, l = token_ids.shape
    vocab, embed = table.shape
    n = b * l
    assert n % (_NUM_WORKERS * _CHUNK) == 0
    n_chunks = n // (_NUM_WORKERS * _CHUNK)
    idx = token_ids.astype(jnp.int32).reshape(_NUM_WORKERS, n_chunks, _CHUNK)
    eyepad = jnp.eye(embed, _LANES, dtype=jnp.float32)
    table_pad = _pad_tc(table.T, eyepad, vocab=vocab, embed=embed)
    out = _gather_sc(idx, table_pad, n_chunks=n_chunks, embed=embed)
    return out.reshape(b, l, embed)


# speedup vs baseline: 4.1084x; 1.0443x over previous
import functools

import jax
import jax.numpy as jnp
from jax import lax
from jax.experimental import pallas as pl
from jax.experimental.pallas import tpu as pltpu
from jax.experimental.pallas import tpu_sc as plsc

_NUM_CORES = 2
_NUM_SUBCORES = 16
_NUM_WORKERS = _NUM_CORES * _NUM_SUBCORES
_CHUNK = 256
_NBUF = 3
_LANES = 128
_BLK = 16384


def _pad_body(t_ref, eye_ref, o_ref):
    del eye_ref
    xt = t_ref[...].T
    o_ref[...] = jnp.concatenate(
        [xt, jnp.zeros((xt.shape[0], _LANES - xt.shape[1]), jnp.float32)], axis=1
    )


@functools.partial(jax.jit, static_argnames=("vocab", "embed"))
def _pad_tc(table_t, eyepad, *, vocab, embed):
    return pl.pallas_call(
        _pad_body,
        grid=(pl.cdiv(vocab, _BLK),),
        in_specs=[
            pl.BlockSpec((embed, _BLK), lambda i: (0, i)),
            pl.BlockSpec((embed, _LANES), lambda i: (0, 0)),
        ],
        out_specs=pl.BlockSpec((_BLK, _LANES), lambda i: (i, 0)),
        out_shape=jax.ShapeDtypeStruct((vocab, _LANES), jnp.float32),
    )(table_t, eyepad)


@functools.partial(jax.jit, static_argnames=("n_chunks", "embed"))
def _gather_sc(idx, table_pad, *, n_chunks, embed):
    mesh = plsc.VectorSubcoreMesh(core_axis_name="c", subcore_axis_name="s")

    @functools.partial(
        pl.kernel,
        out_type=jax.ShapeDtypeStruct(
            (_NUM_WORKERS, n_chunks, _CHUNK, embed), jnp.float32
        ),
        mesh=mesh,
        compiler_params=pltpu.CompilerParams(use_tc_tiling_on_sc=False),
        scratch_types=[
            pltpu.VMEM((n_chunks, _CHUNK), jnp.int32),
            pltpu.VMEM((_NBUF, _CHUNK, _LANES), jnp.float32),
            pltpu.SemaphoreType.DMA((_NBUF,)),
            pltpu.SemaphoreType.DMA((_NBUF,)),
        ],
    )
    def k(idx_hbm, table_hbm, out_hbm, idx_v, rows_v, gsem, osem):
        wid = lax.axis_index("s") * _NUM_CORES + lax.axis_index("c")
        pltpu.sync_copy(idx_hbm.at[wid], idx_v)

        gathers = [None] * n_chunks
        outs = [None] * n_chunks

        def start_out(g):
            b = g % _NBUF
            return pltpu.async_copy(
                rows_v.at[b, slice(None), pl.ds(0, embed)],
                out_hbm.at[wid, g],
                osem.at[b],
            )

        for g in range(n_chunks):
            b = g % _NBUF
            if g >= _NBUF:
                outs[g - _NBUF].wait()
            gathers[g] = pltpu.async_copy(
                table_hbm.at[idx_v.at[g]], rows_v.at[b], gsem.at[b]
            )
            if g >= 1:
                gathers[g - 1].wait()
                outs[g - 1] = start_out(g - 1)
        gathers[n_chunks - 1].wait()
        outs[n_chunks - 1] = start_out(n_chunks - 1)
        for g in range(max(0, n_chunks - _NBUF), n_chunks):
            outs[g].wait()

    return k(idx, table_pad)


def kernel(token_ids, table):
    b, l = token_ids.shape
    vocab, embed = table.shape
    n = b * l
    assert n % (_NUM_WORKERS * _CHUNK) == 0
    n_chunks = n // (_NUM_WORKERS * _CHUNK)
    idx = token_ids.astype(jnp.int32).reshape(_NUM_WORKERS, n_chunks, _CHUNK)
    eyepad = jnp.eye(embed, _LANES, dtype=jnp.float32)
    table_pad = _pad_tc(table.T, eyepad, vocab=vocab, embed=embed)
    out = _gather_sc(idx, table_pad, n_chunks=n_chunks, embed=embed)
    return out.reshape(b, l, embed)
